# baseline (speedup 1.0000x reference)
import functools
import math

import jax
import jax.numpy as jnp
from jax import lax
from jax.experimental import pallas as pl
from jax.experimental.pallas import tpu as pltpu
from jax.experimental.pallas import tpu_sc as plsc

DEPTH_L = 18
NUM_LEAVES = 2 ** DEPTH_L
NUM_NODES_K = NUM_LEAVES - 1
BATCH = 2000000
SCALE = float(NUM_LEAVES)
BONUS = DEPTH_L * math.log(2.0)

NC, NS, LANES = 2, 16, 16
NW = NC * NS

_LOG_C = (0.043428907822139526, -0.4048671744191854, 1.5939013634991297,
          -3.49249427987935, 5.046876044975941, -2.786812953867443)
_LN2 = math.log(2.0)

TPB = NUM_LEAVES // NW
_OFFN = [2 ** (DEPTH_L - 1 - m) - 1 for m in range(DEPTH_L)]
_NSPAN = [max(TPB >> (m + 1), 1) + 1 for m in range(DEPTH_L)]
_H = [(-(-(s + 16) // 16)) * 16 for s in _NSPAN]
_BASE = [2 * sum(_H[:m]) for m in range(DEPTH_L)]
STAGE_TOTAL = 2 * sum(_H)
_LEN0 = 4103
COARSE = TPB // LANES

CHUNK = 4000
NCHUNKS = BATCH // CHUNK
MAX_ITERS = -(-NCHUNKS // NW)

_MESH = plsc.VectorSubcoreMesh(
    core_axis_name="c", subcore_axis_name="s", num_cores=NC, num_subcores=NS)
_PARAMS = pltpu.CompilerParams(
    needs_layout_passes=False, use_tc_tiling_on_sc=False)


def _vlog(v):
    bits = plsc.bitcast(v, jnp.int32)
    e = jnp.right_shift(bits, 23) - 127
    mant = plsc.bitcast(
        jnp.bitwise_or(jnp.bitwise_and(bits, 0x007FFFFF), 0x3F800000),
        jnp.float32)
    acc = mant * _LOG_C[0] + _LOG_C[1]
    for coef in _LOG_C[2:]:
        acc = acc * mant + coef
    return (acc + e.astype(jnp.float32)) * _LN2


@functools.partial(
    pl.kernel,
    out_type=jax.ShapeDtypeStruct((NUM_LEAVES,), jnp.float32),
    mesh=_MESH,
    compiler_params=_PARAMS,
    scratch_types=[
        pltpu.VMEM((STAGE_TOTAL,), jnp.float32),
        pltpu.VMEM((COARSE,), jnp.float32),
        pltpu.VMEM((TPB,), jnp.float32),
        pltpu.SemaphoreType.DMA,
    ],
)
def _build_table(th0_hbm, th1_hbm, s_hbm, stage_v, coarse_v, out_v, sem):
    wid = lax.axis_index("s") * NC + lax.axis_index("c")
    c0 = wid * TPB

    cols = (th0_hbm, th1_hbm)
    descs = []
    adjs = [None] * DEPTH_L
    n0_al = pl.multiple_of(c0 // 2 + 131064, 8)
    for b in (0, 1):
        descs.append(pltpu.async_copy(
            cols[b].at[pl.ds(n0_al, _LEN0)],
            stage_v.at[pl.ds(_BASE[0] + b * _H[0], _LEN0)], sem))
    adjs[0] = _OFFN[0] - n0_al + _BASE[0]
    for m in range(1, DEPTH_L):
        n0 = _OFFN[m] + jnp.right_shift(c0, m + 1)
        na = pl.multiple_of(jnp.bitwise_and(n0, jnp.int32(-8)), 8)
        for b in (0, 1):
            descs.append(pltpu.async_copy(
                cols[b].at[pl.ds(na, _H[m])],
                stage_v.at[pl.ds(_BASE[m] + b * _H[m], _H[m])], sem))
        adjs[m] = _OFFN[m] - na + _BASE[m]
    for d in descs:
        d.wait()

    iota = lax.iota(jnp.int32, LANES)

    @plsc.parallel_loop(0, STAGE_TOTAL // LANES, unroll=4)
    def _log_loop(j):
        sl = stage_v[pl.ds(j * LANES, LANES)]
        stage_v[pl.ds(j * LANES, LANES)] = _vlog(sl + 1e-20)

    def _acc_level(acc, m, sh):
        idx = (jnp.right_shift(sh, 1) + adjs[m]
               + jnp.bitwise_and(sh, 1) * _H[m])
        return acc + plsc.load_gather(stage_v, [idx])

    h0 = jnp.right_shift(c0, 4)

    @plsc.parallel_loop(0, COARSE // LANES, unroll=2)
    def _coarse_loop(u):
        h_vec = h0 + u * LANES + iota
        acc = jnp.full((LANES,), BONUS, jnp.float32)
        for m in range(4, DEPTH_L):
            acc = _acc_level(acc, m, jnp.right_shift(h_vec, m - 4))
        coarse_v[pl.ds(u * LANES, LANES)] = acc

    @plsc.parallel_loop(0, TPB // LANES, unroll=2)
    def _fine_loop(t):
        c_vec = c0 + t * LANES + iota
        acc = plsc.load_gather(coarse_v, [jnp.broadcast_to(t, (LANES,))])
        for m in range(4):
            acc = _acc_level(acc, m, jnp.right_shift(c_vec, m))
        out_v[pl.ds(t * LANES, LANES)] = acc

    pltpu.sync_copy(out_v, s_hbm.at[pl.ds(c0, TPB)])


@functools.partial(
    pl.kernel,
    out_type=jax.ShapeDtypeStruct((BATCH,), jnp.float32),
    mesh=_MESH,
    compiler_params=_PARAMS,
    scratch_types=[
        pltpu.VMEM((CHUNK,), jnp.float32),
        pltpu.VMEM((CHUNK,), jnp.float32),
        pltpu.VMEM((CHUNK,), jnp.int32),
        pltpu.VMEM((CHUNK,), jnp.int32),
        pltpu.VMEM((CHUNK,), jnp.float32),
        pltpu.VMEM((CHUNK,), jnp.float32),
        pltpu.SemaphoreType.DMA,
        pltpu.SemaphoreType.DMA,
        pltpu.SemaphoreType.DMA,
        pltpu.SemaphoreType.DMA,
        pltpu.SemaphoreType.DMA,
    ],
)
def _gather_leaves(x_hbm, s_hbm, out_hbm,
                   x0, x1, i0, i1, r0, r1, sx0, sx1, sg, ss0, ss1):
    wid = lax.axis_index("s") * NC + lax.axis_index("c")
    xs, idxs, rs = (x0, x1), (i0, i1), (r0, r1)
    sxs, sss = (sx0, sx1), (ss0, ss1)

    def chunk_base(k):
        cid = k * NW + wid
        cid = jnp.where(cid < NCHUNKS, cid, cid - NW)
        return pl.multiple_of(cid * CHUNK, 8)

    def idx_compute(b):
        @plsc.parallel_loop(0, CHUNK // LANES, unroll=8)
        def _idx_loop(t):
            xv = xs[b][pl.ds(t * LANES, LANES)]
            ci = (xv * SCALE).astype(jnp.int32)
            ci = jnp.minimum(jnp.maximum(ci, 0), NUM_LEAVES - 1)
            idxs[b][pl.ds(t * LANES, LANES)] = ci

    dx = [None, None]
    dg = [None, None]
    dst = [None, None]
    dx[0] = pltpu.async_copy(
        x_hbm.at[pl.ds(chunk_base(0), CHUNK)], xs[0], sxs[0])
    for k in range(MAX_ITERS):
        b = k & 1
        if k + 1 < MAX_ITERS:
            dx[1 - b] = pltpu.async_copy(
                x_hbm.at[pl.ds(chunk_base(k + 1), CHUNK)], xs[1 - b],
                sxs[1 - b])
        dx[b].wait()
        idx_compute(b)
        if k >= 1:
            dg[1 - b].wait()
            dst[1 - b] = pltpu.async_copy(
                rs[1 - b], out_hbm.at[pl.ds(chunk_base(k - 1), CHUNK)],
                sss[1 - b])
        if k >= 2:
            dst[b].wait()
        dg[b] = pltpu.async_copy(s_hbm.at[idxs[b]], rs[b], sg)
    bl = (MAX_ITERS - 1) & 1
    dg[bl].wait()
    dst[bl] = pltpu.async_copy(
        rs[bl], out_hbm.at[pl.ds(chunk_base(MAX_ITERS - 1), CHUNK)], sss[bl])
    dst[1 - bl].wait()
    dst[bl].wait()


def kernel(x, theta):
    s_table = _build_table(theta[:, 0], theta[:, 1])
    return _gather_leaves(x, s_table)


# speedup vs baseline: 1.0107x; 1.0107x over previous
import functools
import math

import jax
import jax.numpy as jnp
from jax import lax
from jax.experimental import pallas as pl
from jax.experimental.pallas import tpu as pltpu
from jax.experimental.pallas import tpu_sc as plsc

DEPTH_L = 18
NUM_LEAVES = 2 ** DEPTH_L
NUM_NODES_K = NUM_LEAVES - 1
BATCH = 2000000
SCALE = float(NUM_LEAVES)
BONUS = DEPTH_L * math.log(2.0)

NC, NS, LANES = 2, 16, 16
NW = NC * NS

_LOG_C = (0.043428907822139526, -0.4048671744191854, 1.5939013634991297,
          -3.49249427987935, 5.046876044975941, -2.786812953867443)
_LN2 = math.log(2.0)

TPB = NUM_LEAVES // NW
_OFFN = [2 ** (DEPTH_L - 1 - m) - 1 for m in range(DEPTH_L)]
_NSPAN = [max(TPB >> (m + 1), 1) + 1 for m in range(DEPTH_L)]
_H = [(-(-(s + 16) // 16)) * 16 for s in _NSPAN]
_BASE = [2 * sum(_H[:m]) for m in range(DEPTH_L)]
STAGE_TOTAL = 2 * sum(_H)
_LEN0 = 4103
COARSE = TPB // LANES

CHUNK = 8000
NCHUNKS = BATCH // CHUNK
MAX_ITERS = -(-NCHUNKS // NW)

_MESH = plsc.VectorSubcoreMesh(
    core_axis_name="c", subcore_axis_name="s", num_cores=NC, num_subcores=NS)
_PARAMS = pltpu.CompilerParams(
    needs_layout_passes=False, use_tc_tiling_on_sc=False)


def _vlog(v):
    bits = plsc.bitcast(v, jnp.int32)
    e = jnp.right_shift(bits, 23) - 127
    mant = plsc.bitcast(
        jnp.bitwise_or(jnp.bitwise_and(bits, 0x007FFFFF), 0x3F800000),
        jnp.float32)
    acc = mant * _LOG_C[0] + _LOG_C[1]
    for coef in _LOG_C[2:]:
        acc = acc * mant + coef
    return (acc + e.astype(jnp.float32)) * _LN2


@functools.partial(
    pl.kernel,
    out_type=jax.ShapeDtypeStruct((NUM_LEAVES,), jnp.float32),
    mesh=_MESH,
    compiler_params=_PARAMS,
    scratch_types=[
        pltpu.VMEM((STAGE_TOTAL,), jnp.float32),
        pltpu.VMEM((COARSE,), jnp.float32),
        pltpu.VMEM((TPB,), jnp.float32),
        pltpu.SemaphoreType.DMA,
    ],
)
def _build_table(th0_hbm, th1_hbm, s_hbm, stage_v, coarse_v, out_v, sem):
    wid = lax.axis_index("s") * NC + lax.axis_index("c")
    c0 = wid * TPB

    cols = (th0_hbm, th1_hbm)
    descs = []
    adjs = [None] * DEPTH_L
    n0_al = pl.multiple_of(c0 // 2 + 131064, 8)
    for b in (0, 1):
        descs.append(pltpu.async_copy(
            cols[b].at[pl.ds(n0_al, _LEN0)],
            stage_v.at[pl.ds(_BASE[0] + b * _H[0], _LEN0)], sem))
    adjs[0] = _OFFN[0] - n0_al + _BASE[0]
    for m in range(1, DEPTH_L):
        n0 = _OFFN[m] + jnp.right_shift(c0, m + 1)
        na = pl.multiple_of(jnp.bitwise_and(n0, jnp.int32(-8)), 8)
        for b in (0, 1):
            descs.append(pltpu.async_copy(
                cols[b].at[pl.ds(na, _H[m])],
                stage_v.at[pl.ds(_BASE[m] + b * _H[m], _H[m])], sem))
        adjs[m] = _OFFN[m] - na + _BASE[m]
    for d in descs:
        d.wait()

    iota = lax.iota(jnp.int32, LANES)

    @plsc.parallel_loop(0, STAGE_TOTAL // LANES, unroll=4)
    def _log_loop(j):
        sl = stage_v[pl.ds(j * LANES, LANES)]
        stage_v[pl.ds(j * LANES, LANES)] = _vlog(sl + 1e-20)

    def _acc_level(acc, m, sh):
        idx = (jnp.right_shift(sh, 1) + adjs[m]
               + jnp.bitwise_and(sh, 1) * _H[m])
        return acc + plsc.load_gather(stage_v, [idx])

    h0 = jnp.right_shift(c0, 4)

    @plsc.parallel_loop(0, COARSE // LANES, unroll=2)
    def _coarse_loop(u):
        h_vec = h0 + u * LANES + iota
        acc = jnp.full((LANES,), BONUS, jnp.float32)
        for m in range(4, DEPTH_L):
            acc = _acc_level(acc, m, jnp.right_shift(h_vec, m - 4))
        coarse_v[pl.ds(u * LANES, LANES)] = acc

    @plsc.parallel_loop(0, TPB // LANES, unroll=2)
    def _fine_loop(t):
        c_vec = c0 + t * LANES + iota
        acc = plsc.load_gather(coarse_v, [jnp.broadcast_to(t, (LANES,))])
        for m in range(4):
            acc = _acc_level(acc, m, jnp.right_shift(c_vec, m))
        out_v[pl.ds(t * LANES, LANES)] = acc

    pltpu.sync_copy(out_v, s_hbm.at[pl.ds(c0, TPB)])


@functools.partial(
    pl.kernel,
    out_type=jax.ShapeDtypeStruct((BATCH,), jnp.float32),
    mesh=_MESH,
    compiler_params=_PARAMS,
    scratch_types=[
        pltpu.VMEM((CHUNK,), jnp.float32),
        pltpu.VMEM((CHUNK,), jnp.float32),
        pltpu.VMEM((CHUNK,), jnp.int32),
        pltpu.VMEM((CHUNK,), jnp.int32),
        pltpu.VMEM((CHUNK,), jnp.float32),
        pltpu.VMEM((CHUNK,), jnp.float32),
        pltpu.SemaphoreType.DMA,
        pltpu.SemaphoreType.DMA,
        pltpu.SemaphoreType.DMA,
        pltpu.SemaphoreType.DMA,
        pltpu.SemaphoreType.DMA,
    ],
)
def _gather_leaves(x_hbm, s_hbm, out_hbm,
                   x0, x1, i0, i1, r0, r1, sx0, sx1, sg, ss0, ss1):
    wid = lax.axis_index("s") * NC + lax.axis_index("c")
    xs, idxs, rs = (x0, x1), (i0, i1), (r0, r1)
    sxs, sss = (sx0, sx1), (ss0, ss1)

    def chunk_base(k):
        cid = k * NW + wid
        cid = jnp.where(cid < NCHUNKS, cid, cid - NW)
        return pl.multiple_of(cid * CHUNK, 8)

    def idx_compute(b):
        @plsc.parallel_loop(0, CHUNK // LANES, unroll=8)
        def _idx_loop(t):
            xv = xs[b][pl.ds(t * LANES, LANES)]
            ci = (xv * SCALE).astype(jnp.int32)
            ci = jnp.minimum(jnp.maximum(ci, 0), NUM_LEAVES - 1)
            idxs[b][pl.ds(t * LANES, LANES)] = ci

    dx = [None, None]
    dg = [None, None]
    dst = [None, None]
    dx[0] = pltpu.async_copy(
        x_hbm.at[pl.ds(chunk_base(0), CHUNK)], xs[0], sxs[0])
    for k in range(MAX_ITERS):
        b = k & 1
        if k + 1 < MAX_ITERS:
            dx[1 - b] = pltpu.async_copy(
                x_hbm.at[pl.ds(chunk_base(k + 1), CHUNK)], xs[1 - b],
                sxs[1 - b])
        dx[b].wait()
        idx_compute(b)
        if k >= 1:
            dg[1 - b].wait()
            dst[1 - b] = pltpu.async_copy(
                rs[1 - b], out_hbm.at[pl.ds(chunk_base(k - 1), CHUNK)],
                sss[1 - b])
        if k >= 2:
            dst[b].wait()
        dg[b] = pltpu.async_copy(s_hbm.at[idxs[b]], rs[b], sg)
    bl = (MAX_ITERS - 1) & 1
    dg[bl].wait()
    dst[bl] = pltpu.async_copy(
        rs[bl], out_hbm.at[pl.ds(chunk_base(MAX_ITERS - 1), CHUNK)], sss[bl])
    dst[1 - bl].wait()
    dst[bl].wait()


def kernel(x, theta):
    s_table = _build_table(theta[:, 0], theta[:, 1])
    return _gather_leaves(x, s_table)


# speedup vs baseline: 2.0378x; 2.0162x over previous
import functools
import math

import jax
import jax.numpy as jnp
from jax import lax
from jax.experimental import pallas as pl
from jax.experimental.pallas import tpu as pltpu
from jax.experimental.pallas import tpu_sc as plsc

DEPTH_L = 18
NUM_LEAVES = 2 ** DEPTH_L
NUM_NODES_K = NUM_LEAVES - 1
BATCH = 2000000
SCALE = float(NUM_LEAVES)
BONUS = DEPTH_L * math.log(2.0)

NC, NS, LANES = 2, 16, 16
NW = NC * NS

_LOG_C = (0.043428907822139526, -0.4048671744191854, 1.5939013634991297,
          -3.49249427987935, 5.046876044975941, -2.786812953867443)
_LN2 = math.log(2.0)

TPB = NUM_LEAVES // NW
_OFFN = [2 ** (DEPTH_L - 1 - m) - 1 for m in range(DEPTH_L)]
_NSPAN = [max(TPB >> (m + 1), 1) + 1 for m in range(DEPTH_L)]
_H = [(-(-(s + 16) // 16)) * 16 for s in _NSPAN]
_BASE = [2 * sum(_H[:m]) for m in range(DEPTH_L)]
STAGE_TOTAL = 2 * sum(_H)
_LEN0 = 4103
COARSE = TPB // LANES

CHUNK = 8000
NCHUNKS = BATCH // CHUNK
MAX_ITERS = -(-NCHUNKS // NW)

_MESH = plsc.VectorSubcoreMesh(
    core_axis_name="c", subcore_axis_name="s", num_cores=NC, num_subcores=NS)
_PARAMS = pltpu.CompilerParams(
    needs_layout_passes=False, use_tc_tiling_on_sc=False)


def _vlog(v):
    bits = plsc.bitcast(v, jnp.int32)
    e = jnp.right_shift(bits, 23) - 127
    mant = plsc.bitcast(
        jnp.bitwise_or(jnp.bitwise_and(bits, 0x007FFFFF), 0x3F800000),
        jnp.float32)
    acc = mant * _LOG_C[0] + _LOG_C[1]
    for coef in _LOG_C[2:]:
        acc = acc * mant + coef
    return (acc + e.astype(jnp.float32)) * _LN2


@functools.partial(
    pl.kernel,
    out_type=jax.ShapeDtypeStruct((NUM_LEAVES,), jnp.float32),
    mesh=_MESH,
    compiler_params=_PARAMS,
    scratch_types=[
        pltpu.VMEM((STAGE_TOTAL,), jnp.float32),
        pltpu.VMEM((COARSE,), jnp.float32),
        pltpu.VMEM((TPB,), jnp.float32),
        pltpu.SemaphoreType.DMA,
    ],
)
def _build_table(th0_hbm, th1_hbm, s_hbm, stage_v, coarse_v, out_v, sem):
    wid = lax.axis_index("s") * NC + lax.axis_index("c")
    c0 = wid * TPB

    cols = (th0_hbm, th1_hbm)
    descs = []
    adjs = [None] * DEPTH_L
    n0_al = pl.multiple_of(c0 // 2 + 131064, 8)
    for b in (0, 1):
        descs.append(pltpu.async_copy(
            cols[b].at[pl.ds(n0_al, _LEN0)],
            stage_v.at[pl.ds(_BASE[0] + b * _H[0], _LEN0)], sem))
    adjs[0] = _OFFN[0] - n0_al + _BASE[0]
    for m in range(1, DEPTH_L):
        n0 = _OFFN[m] + jnp.right_shift(c0, m + 1)
        na = pl.multiple_of(jnp.bitwise_and(n0, jnp.int32(-8)), 8)
        for b in (0, 1):
            descs.append(pltpu.async_copy(
                cols[b].at[pl.ds(na, _H[m])],
                stage_v.at[pl.ds(_BASE[m] + b * _H[m], _H[m])], sem))
        adjs[m] = _OFFN[m] - na + _BASE[m]
    for d in descs:
        d.wait()

    iota = lax.iota(jnp.int32, LANES)

    @plsc.parallel_loop(0, STAGE_TOTAL // LANES, unroll=4)
    def _log_loop(j):
        sl = stage_v[pl.ds(j * LANES, LANES)]
        stage_v[pl.ds(j * LANES, LANES)] = _vlog(sl + 1e-20)

    def _acc_level(acc, m, sh):
        idx = (jnp.right_shift(sh, 1) + adjs[m]
               + jnp.bitwise_and(sh, 1) * _H[m])
        return acc + plsc.load_gather(stage_v, [idx])

    h0 = jnp.right_shift(c0, 4)

    @plsc.parallel_loop(0, COARSE // LANES, unroll=2)
    def _coarse_loop(u):
        h_vec = h0 + u * LANES + iota
        acc = jnp.full((LANES,), BONUS, jnp.float32)
        for m in range(4, DEPTH_L):
            acc = _acc_level(acc, m, jnp.right_shift(h_vec, m - 4))
        coarse_v[pl.ds(u * LANES, LANES)] = acc

    @plsc.parallel_loop(0, TPB // LANES, unroll=2)
    def _fine_loop(t):
        c_vec = c0 + t * LANES + iota
        acc = plsc.load_gather(coarse_v, [jnp.broadcast_to(t, (LANES,))])
        for m in range(4):
            acc = _acc_level(acc, m, jnp.right_shift(c_vec, m))
        out_v[pl.ds(t * LANES, LANES)] = acc

    pltpu.sync_copy(out_v, s_hbm.at[pl.ds(c0, TPB)])


@functools.partial(
    pl.kernel,
    out_type=jax.ShapeDtypeStruct((BATCH,), jnp.float32),
    mesh=_MESH,
    compiler_params=_PARAMS,
    scratch_types=[
        pltpu.VMEM((CHUNK,), jnp.float32),
        pltpu.VMEM((CHUNK,), jnp.float32),
        pltpu.VMEM((CHUNK,), jnp.int32),
        pltpu.VMEM((CHUNK,), jnp.int32),
        pltpu.VMEM((CHUNK,), jnp.float32),
        pltpu.VMEM((CHUNK,), jnp.float32),
        pltpu.VMEM_SHARED((NUM_LEAVES,), jnp.float32),
        pltpu.SemaphoreType.DMA,
        pltpu.SemaphoreType.DMA,
        pltpu.SemaphoreType.DMA,
        pltpu.SemaphoreType.DMA,
        pltpu.SemaphoreType.DMA,
    ],
)
def _gather_leaves(x_hbm, s_hbm, out_hbm,
                   x0, x1, i0, i1, r0, r1, s_sh, sx0, sx1, sg, ss0, ss1):
    wid = lax.axis_index("s") * NC + lax.axis_index("c")
    sid = lax.axis_index("s")
    xs, idxs, rs = (x0, x1), (i0, i1), (r0, r1)
    sxs, sss = (sx0, sx1), (ss0, ss1)

    seg = NUM_LEAVES // NS
    pltpu.sync_copy(s_hbm.at[pl.ds(sid * seg, seg)],
                    s_sh.at[pl.ds(sid * seg, seg)])
    plsc.subcore_barrier()

    def chunk_base(k):
        cid = k * NW + wid
        cid = jnp.where(cid < NCHUNKS, cid, cid - NW)
        return pl.multiple_of(cid * CHUNK, 8)

    def idx_compute(b):
        @plsc.parallel_loop(0, CHUNK // LANES, unroll=8)
        def _idx_loop(t):
            xv = xs[b][pl.ds(t * LANES, LANES)]
            ci = (xv * SCALE).astype(jnp.int32)
            ci = jnp.minimum(jnp.maximum(ci, 0), NUM_LEAVES - 1)
            idxs[b][pl.ds(t * LANES, LANES)] = ci

    dx = [None, None]
    dg = [None, None]
    dst = [None, None]
    dx[0] = pltpu.async_copy(
        x_hbm.at[pl.ds(chunk_base(0), CHUNK)], xs[0], sxs[0])
    for k in range(MAX_ITERS):
        b = k & 1
        if k + 1 < MAX_ITERS:
            dx[1 - b] = pltpu.async_copy(
                x_hbm.at[pl.ds(chunk_base(k + 1), CHUNK)], xs[1 - b],
                sxs[1 - b])
        dx[b].wait()
        idx_compute(b)
        if k >= 1:
            dg[1 - b].wait()
            dst[1 - b] = pltpu.async_copy(
                rs[1 - b], out_hbm.at[pl.ds(chunk_base(k - 1), CHUNK)],
                sss[1 - b])
        if k >= 2:
            dst[b].wait()
        dg[b] = pltpu.async_copy(s_sh.at[idxs[b]], rs[b], sg)
    bl = (MAX_ITERS - 1) & 1
    dg[bl].wait()
    dst[bl] = pltpu.async_copy(
        rs[bl], out_hbm.at[pl.ds(chunk_base(MAX_ITERS - 1), CHUNK)], sss[bl])
    dst[1 - bl].wait()
    dst[bl].wait()


def kernel(x, theta):
    s_table = _build_table(theta[:, 0], theta[:, 1])
    return _gather_leaves(x, s_table)


# speedup vs baseline: 2.0603x; 1.0110x over previous
import functools
import math

import jax
import jax.numpy as jnp
from jax import lax
from jax.experimental import pallas as pl
from jax.experimental.pallas import tpu as pltpu
from jax.experimental.pallas import tpu_sc as plsc

DEPTH_L = 18
NUM_LEAVES = 2 ** DEPTH_L
NUM_NODES_K = NUM_LEAVES - 1
BATCH = 2000000
SCALE = float(NUM_LEAVES)
BONUS = DEPTH_L * math.log(2.0)

NC, NS, LANES = 2, 16, 16
NW = NC * NS

_LOG_C = (0.043428907822139526, -0.4048671744191854, 1.5939013634991297,
          -3.49249427987935, 5.046876044975941, -2.786812953867443)
_LN2 = math.log(2.0)

TPB = NUM_LEAVES // NW
_OFFN = [2 ** (DEPTH_L - 1 - m) - 1 for m in range(DEPTH_L)]
_NLEV = 8
_NSPAN = [max(TPB >> (m + 1), 1) + 1 for m in range(_NLEV)]
_H = [(-(-(s + 16) // 16)) * 16 for s in _NSPAN]
_BASE = [2 * sum(_H[:m]) for m in range(_NLEV)]
_TOPB = 2 * sum(_H)
_TOPH = 1024
STAGE_TOTAL = _TOPB + 2 * _TOPH
_LEN0 = 4103
COARSE = TPB // LANES

CHUNK = 8000
NCHUNKS = BATCH // CHUNK
MAX_ITERS = -(-NCHUNKS // NW)

_MESH = plsc.VectorSubcoreMesh(
    core_axis_name="c", subcore_axis_name="s", num_cores=NC, num_subcores=NS)
_PARAMS = pltpu.CompilerParams(
    needs_layout_passes=False, use_tc_tiling_on_sc=False)


def _vlog(v):
    bits = plsc.bitcast(v, jnp.int32)
    e = jnp.right_shift(bits, 23) - 127
    mant = plsc.bitcast(
        jnp.bitwise_or(jnp.bitwise_and(bits, 0x007FFFFF), 0x3F800000),
        jnp.float32)
    acc = mant * _LOG_C[0] + _LOG_C[1]
    for coef in _LOG_C[2:]:
        acc = acc * mant + coef
    return (acc + e.astype(jnp.float32)) * _LN2


@functools.partial(
    pl.kernel,
    out_type=jax.ShapeDtypeStruct((NUM_LEAVES,), jnp.float32),
    mesh=_MESH,
    compiler_params=_PARAMS,
    scratch_types=[
        pltpu.VMEM((STAGE_TOTAL,), jnp.float32),
        pltpu.VMEM((COARSE,), jnp.float32),
        pltpu.VMEM((TPB,), jnp.float32),
        pltpu.SemaphoreType.DMA,
    ],
)
def _build_table(th0_hbm, th1_hbm, s_hbm, stage_v, coarse_v, out_v, sem):
    wid = lax.axis_index("s") * NC + lax.axis_index("c")
    c0 = wid * TPB

    cols = (th0_hbm, th1_hbm)
    descs = []
    adjs = [None] * _NLEV
    n0_al = pl.multiple_of(c0 // 2 + 131064, 8)
    for b in (0, 1):
        descs.append(pltpu.async_copy(
            cols[b].at[pl.ds(n0_al, _LEN0)],
            stage_v.at[pl.ds(_BASE[0] + b * _H[0], _LEN0)], sem))
    adjs[0] = _OFFN[0] - n0_al + _BASE[0]
    for m in range(1, _NLEV):
        n0 = _OFFN[m] + jnp.right_shift(c0, m + 1)
        na = pl.multiple_of(jnp.bitwise_and(n0, jnp.int32(-8)), 8)
        for b in (0, 1):
            descs.append(pltpu.async_copy(
                cols[b].at[pl.ds(na, _H[m])],
                stage_v.at[pl.ds(_BASE[m] + b * _H[m], _H[m])], sem))
        adjs[m] = _OFFN[m] - na + _BASE[m]
    for b in (0, 1):
        descs.append(pltpu.async_copy(
            cols[b].at[pl.ds(0, _TOPH)],
            stage_v.at[pl.ds(_TOPB + b * _TOPH, _TOPH)], sem))
    for d in descs:
        d.wait()

    iota = lax.iota(jnp.int32, LANES)

    @plsc.parallel_loop(0, STAGE_TOTAL // LANES, unroll=8)
    def _log_loop(j):
        sl = stage_v[pl.ds(j * LANES, LANES)]
        stage_v[pl.ds(j * LANES, LANES)] = _vlog(sl + 1e-20)

    def _acc_level(acc, m, sh):
        if m < _NLEV:
            idx = (jnp.right_shift(sh, 1) + adjs[m]
                   + jnp.bitwise_and(sh, 1) * _H[m])
        else:
            idx = (jnp.right_shift(sh, 1) + (_TOPB + _OFFN[m])
                   + jnp.bitwise_and(sh, 1) * _TOPH)
        return acc + plsc.load_gather(stage_v, [idx])

    h0 = jnp.right_shift(c0, 4)

    @plsc.parallel_loop(0, COARSE // LANES, unroll=2)
    def _coarse_loop(u):
        h_vec = h0 + u * LANES + iota
        acc = jnp.full((LANES,), BONUS, jnp.float32)
        for m in range(4, DEPTH_L):
            acc = _acc_level(acc, m, jnp.right_shift(h_vec, m - 4))
        coarse_v[pl.ds(u * LANES, LANES)] = acc

    @plsc.parallel_loop(0, TPB // LANES, unroll=2)
    def _fine_loop(t):
        c_vec = c0 + t * LANES + iota
        acc = plsc.load_gather(coarse_v, [jnp.broadcast_to(t, (LANES,))])
        for m in range(4):
            acc = _acc_level(acc, m, jnp.right_shift(c_vec, m))
        out_v[pl.ds(t * LANES, LANES)] = acc

    pltpu.sync_copy(out_v, s_hbm.at[pl.ds(c0, TPB)])


@functools.partial(
    pl.kernel,
    out_type=jax.ShapeDtypeStruct((BATCH,), jnp.float32),
    mesh=_MESH,
    compiler_params=_PARAMS,
    scratch_types=[
        pltpu.VMEM((CHUNK,), jnp.float32),
        pltpu.VMEM((CHUNK,), jnp.float32),
        pltpu.VMEM((CHUNK,), jnp.int32),
        pltpu.VMEM((CHUNK,), jnp.int32),
        pltpu.VMEM((CHUNK,), jnp.float32),
        pltpu.VMEM((CHUNK,), jnp.float32),
        pltpu.VMEM_SHARED((NUM_LEAVES,), jnp.float32),
        pltpu.SemaphoreType.DMA,
        pltpu.SemaphoreType.DMA,
        pltpu.SemaphoreType.DMA,
        pltpu.SemaphoreType.DMA,
        pltpu.SemaphoreType.DMA,
    ],
)
def _gather_leaves(x_hbm, s_hbm, out_hbm,
                   x0, x1, i0, i1, r0, r1, s_sh, sx0, sx1, sg, ss0, ss1):
    wid = lax.axis_index("s") * NC + lax.axis_index("c")
    sid = lax.axis_index("s")
    xs, idxs, rs = (x0, x1), (i0, i1), (r0, r1)
    sxs, sss = (sx0, sx1), (ss0, ss1)

    seg = NUM_LEAVES // NS
    pltpu.sync_copy(s_hbm.at[pl.ds(sid * seg, seg)],
                    s_sh.at[pl.ds(sid * seg, seg)])
    plsc.subcore_barrier()

    def chunk_base(k):
        cid = k * NW + wid
        cid = jnp.where(cid < NCHUNKS, cid, cid - NW)
        return pl.multiple_of(cid * CHUNK, 8)

    def idx_compute(b):
        @plsc.parallel_loop(0, CHUNK // LANES, unroll=8)
        def _idx_loop(t):
            xv = xs[b][pl.ds(t * LANES, LANES)]
            ci = (xv * SCALE).astype(jnp.int32)
            ci = jnp.minimum(jnp.maximum(ci, 0), NUM_LEAVES - 1)
            idxs[b][pl.ds(t * LANES, LANES)] = ci

    dx = [None, None]
    dg = [None, None]
    dst = [None, None]
    dx[0] = pltpu.async_copy(
        x_hbm.at[pl.ds(chunk_base(0), CHUNK)], xs[0], sxs[0])
    for k in range(MAX_ITERS):
        b = k & 1
        if k + 1 < MAX_ITERS:
            dx[1 - b] = pltpu.async_copy(
                x_hbm.at[pl.ds(chunk_base(k + 1), CHUNK)], xs[1 - b],
                sxs[1 - b])
        dx[b].wait()
        idx_compute(b)
        if k >= 1:
            dg[1 - b].wait()
            dst[1 - b] = pltpu.async_copy(
                rs[1 - b], out_hbm.at[pl.ds(chunk_base(k - 1), CHUNK)],
                sss[1 - b])
        if k >= 2:
            dst[b].wait()
        dg[b] = pltpu.async_copy(s_sh.at[idxs[b]], rs[b], sg)
    bl = (MAX_ITERS - 1) & 1
    dg[bl].wait()
    dst[bl] = pltpu.async_copy(
        rs[bl], out_hbm.at[pl.ds(chunk_base(MAX_ITERS - 1), CHUNK)], sss[bl])
    dst[1 - bl].wait()
    dst[bl].wait()


def kernel(x, theta):
    s_table = _build_table(theta[:, 0], theta[:, 1])
    return _gather_leaves(x, s_table)


# speedup vs baseline: 2.1954x; 1.0656x over previous
import functools
import math

import jax
import jax.numpy as jnp
from jax import lax
from jax.experimental import pallas as pl
from jax.experimental.pallas import tpu as pltpu
from jax.experimental.pallas import tpu_sc as plsc

DEPTH_L = 18
NUM_LEAVES = 2 ** DEPTH_L
NUM_NODES_K = NUM_LEAVES - 1
BATCH = 2000000
SCALE = float(NUM_LEAVES)
BONUS = DEPTH_L * math.log(2.0)

NC, NS, LANES = 2, 16, 16
NW = NC * NS

_LOG_C = (0.043428907822139526, -0.4048671744191854, 1.5939013634991297,
          -3.49249427987935, 5.046876044975941, -2.786812953867443)
_LN2 = math.log(2.0)

TPB = NUM_LEAVES // NW
_OFFN = [2 ** (DEPTH_L - 1 - m) - 1 for m in range(DEPTH_L)]
_NLEV = 8
_NSPAN = [max(TPB >> (m + 1), 1) + 1 for m in range(_NLEV)]
_H = [(-(-(s + 16) // 16)) * 16 for s in _NSPAN]
_BASE = [2 * sum(_H[:m]) for m in range(_NLEV)]
_TOPB = 2 * sum(_H)
_TOPH = 1024
STAGE_TOTAL = _TOPB + 2 * _TOPH
_LEN0 = 4103
COARSE = TPB // LANES

CHUNK = 8000
NCHUNKS = BATCH // CHUNK
MAX_ITERS = -(-NCHUNKS // NW)

_MESH = plsc.VectorSubcoreMesh(
    core_axis_name="c", subcore_axis_name="s", num_cores=NC, num_subcores=NS)
_PARAMS = pltpu.CompilerParams(
    needs_layout_passes=False, use_tc_tiling_on_sc=False)


def _vlog(v):
    bits = plsc.bitcast(v, jnp.int32)
    e = jnp.right_shift(bits, 23) - 127
    mant = plsc.bitcast(
        jnp.bitwise_or(jnp.bitwise_and(bits, 0x007FFFFF), 0x3F800000),
        jnp.float32)
    acc = mant * _LOG_C[0] + _LOG_C[1]
    for coef in _LOG_C[2:]:
        acc = acc * mant + coef
    return (acc + e.astype(jnp.float32)) * _LN2


@functools.partial(
    pl.kernel,
    out_type=jax.ShapeDtypeStruct((NUM_LEAVES,), jnp.float32),
    mesh=_MESH,
    compiler_params=_PARAMS,
    scratch_types=[
        pltpu.VMEM((STAGE_TOTAL,), jnp.float32),
        pltpu.VMEM((STAGE_TOTAL,), jnp.float32),
        pltpu.VMEM((COARSE,), jnp.float32),
        pltpu.VMEM((TPB,), jnp.float32),
        pltpu.SemaphoreType.DMA,
    ],
)
def _build_table(th0_hbm, th1_hbm, s_hbm, stage_v, inter_v, coarse_v, out_v,
                 sem):
    wid = lax.axis_index("s") * NC + lax.axis_index("c")
    c0 = wid * TPB

    cols = (th0_hbm, th1_hbm)
    descs = []
    nas = [None] * _NLEV
    nas[0] = pl.multiple_of(c0 // 2 + 131064, 8)
    for b in (0, 1):
        descs.append(pltpu.async_copy(
            cols[b].at[pl.ds(nas[0], _LEN0)],
            stage_v.at[pl.ds(_BASE[0] + b * _H[0], _LEN0)], sem))
    for m in range(1, _NLEV):
        n0 = _OFFN[m] + jnp.right_shift(c0, m + 1)
        nas[m] = pl.multiple_of(jnp.bitwise_and(n0, jnp.int32(-8)), 8)
        for b in (0, 1):
            descs.append(pltpu.async_copy(
                cols[b].at[pl.ds(nas[m], _H[m])],
                stage_v.at[pl.ds(_BASE[m] + b * _H[m], _H[m])], sem))
    for b in (0, 1):
        descs.append(pltpu.async_copy(
            cols[b].at[pl.ds(0, _TOPH)],
            stage_v.at[pl.ds(_TOPB + b * _TOPH, _TOPH)], sem))
    for d in descs:
        d.wait()

    iota = lax.iota(jnp.int32, LANES)

    for base, half in [(_BASE[m], _H[m]) for m in range(_NLEV)] + [
            (_TOPB, _TOPH)]:
        @plsc.parallel_loop(0, half // LANES, unroll=4)
        def _log_loop(j, base=base, half=half):
            j16 = j * LANES
            v0 = stage_v[pl.ds(base + j16, LANES)]
            v1 = stage_v[pl.ds(base + half + j16, LANES)]
            pos = base + 2 * j16 + 2 * iota
            plsc.store_scatter(inter_v, [pos], _vlog(v0 + 1e-20))
            plsc.store_scatter(inter_v, [pos + 1], _vlog(v1 + 1e-20))

    iadjs = [_BASE[m] + 2 * _OFFN[m] - 2 * nas[m] for m in range(_NLEV)]
    iadjs += [_TOPB + 2 * _OFFN[m] for m in range(_NLEV, DEPTH_L)]

    def _acc_level(acc, m, sh):
        return acc + plsc.load_gather(inter_v, [sh + iadjs[m]])

    h0 = jnp.right_shift(c0, 4)

    @plsc.parallel_loop(0, COARSE // LANES, unroll=2)
    def _coarse_loop(u):
        h_vec = h0 + u * LANES + iota
        acc = jnp.full((LANES,), BONUS, jnp.float32)
        for m in range(4, DEPTH_L):
            acc = _acc_level(acc, m, jnp.right_shift(h_vec, m - 4))
        coarse_v[pl.ds(u * LANES, LANES)] = acc

    @plsc.parallel_loop(0, TPB // LANES, unroll=2)
    def _fine_loop(t):
        c_vec = c0 + t * LANES + iota
        acc = plsc.load_gather(coarse_v, [jnp.broadcast_to(t, (LANES,))])
        for m in range(4):
            acc = _acc_level(acc, m, jnp.right_shift(c_vec, m))
        out_v[pl.ds(t * LANES, LANES)] = acc

    pltpu.sync_copy(out_v, s_hbm.at[pl.ds(c0, TPB)])


@functools.partial(
    pl.kernel,
    out_type=jax.ShapeDtypeStruct((BATCH,), jnp.float32),
    mesh=_MESH,
    compiler_params=_PARAMS,
    scratch_types=[
        pltpu.VMEM((CHUNK,), jnp.float32),
        pltpu.VMEM((CHUNK,), jnp.float32),
        pltpu.VMEM((CHUNK,), jnp.int32),
        pltpu.VMEM((CHUNK,), jnp.int32),
        pltpu.VMEM((CHUNK,), jnp.float32),
        pltpu.VMEM((CHUNK,), jnp.float32),
        pltpu.VMEM_SHARED((NUM_LEAVES,), jnp.float32),
        pltpu.SemaphoreType.DMA,
        pltpu.SemaphoreType.DMA,
        pltpu.SemaphoreType.DMA,
        pltpu.SemaphoreType.DMA,
        pltpu.SemaphoreType.DMA,
    ],
)
def _gather_leaves(x_hbm, s_hbm, out_hbm,
                   x0, x1, i0, i1, r0, r1, s_sh, sx0, sx1, sg, ss0, ss1):
    wid = lax.axis_index("s") * NC + lax.axis_index("c")
    sid = lax.axis_index("s")
    xs, idxs, rs = (x0, x1), (i0, i1), (r0, r1)
    sxs, sss = (sx0, sx1), (ss0, ss1)

    seg = NUM_LEAVES // NS
    pltpu.sync_copy(s_hbm.at[pl.ds(sid * seg, seg)],
                    s_sh.at[pl.ds(sid * seg, seg)])
    plsc.subcore_barrier()

    def chunk_base(k):
        cid = k * NW + wid
        cid = jnp.where(cid < NCHUNKS, cid, cid - NW)
        return pl.multiple_of(cid * CHUNK, 8)

    def idx_compute(b):
        @plsc.parallel_loop(0, CHUNK // LANES, unroll=8)
        def _idx_loop(t):
            xv = xs[b][pl.ds(t * LANES, LANES)]
            ci = (xv * SCALE).astype(jnp.int32)
            ci = jnp.minimum(jnp.maximum(ci, 0), NUM_LEAVES - 1)
            idxs[b][pl.ds(t * LANES, LANES)] = ci

    dx = [None, None]
    dg = [None, None]
    dst = [None, None]
    dx[0] = pltpu.async_copy(
        x_hbm.at[pl.ds(chunk_base(0), CHUNK)], xs[0], sxs[0])
    for k in range(MAX_ITERS):
        b = k & 1
        if k + 1 < MAX_ITERS:
            dx[1 - b] = pltpu.async_copy(
                x_hbm.at[pl.ds(chunk_base(k + 1), CHUNK)], xs[1 - b],
                sxs[1 - b])
        dx[b].wait()
        idx_compute(b)
        if k >= 1:
            dg[1 - b].wait()
            dst[1 - b] = pltpu.async_copy(
                rs[1 - b], out_hbm.at[pl.ds(chunk_base(k - 1), CHUNK)],
                sss[1 - b])
        if k >= 2:
            dst[b].wait()
        dg[b] = pltpu.async_copy(s_sh.at[idxs[b]], rs[b], sg)
    bl = (MAX_ITERS - 1) & 1
    dg[bl].wait()
    dst[bl] = pltpu.async_copy(
        rs[bl], out_hbm.at[pl.ds(chunk_base(MAX_ITERS - 1), CHUNK)], sss[bl])
    dst[1 - bl].wait()
    dst[bl].wait()


def kernel(x, theta):
    s_table = _build_table(theta[:, 0], theta[:, 1])
    return _gather_leaves(x, s_table)
